# baseline (speedup 1.0000x reference)
import functools

import jax
import jax.numpy as jnp
from jax import lax
from jax.experimental import pallas as pl
from jax.experimental.pallas import tpu as pltpu
from jax.experimental.pallas import tpu_sc as plsc

N = 10000
EMB = 128
HALF = 64
NREL = 2
NB = 5
BN = N // NB
CHUNK = 128
NT = 16
NCH = 158
ACC_ROWS = 20480
TRASH = 2 * N
W1 = 80
W2 = 64


def _leaky(x):
    return jnp.where(x >= 0, x, 0.01 * x)


def _dot(a, b):
    return jnp.dot(a, b, preferred_element_type=jnp.float32)


def _k1_body(des, tweet, wd, bd, wt, bt, wi, bi, wrel, wroot, brgcn,
             ylo, yhi, root):
    d = _leaky(_dot(des[...], wd[...]) + bd[...])
    t = _leaky(_dot(tweet[...], wt[...]) + bt[...])
    x = _leaky(_dot(jnp.concatenate([d, t], axis=1), wi[...]) + bi[...])
    root[...] = _dot(x, wroot[...]) + brgcn[...]
    onecol = (lax.broadcasted_iota(jnp.int32, (BN, W1 - HALF), 1) == 0
              ).astype(jnp.float32)
    for r in range(NREL):
        y = _dot(x, wrel[r])
        ylo[r] = jnp.concatenate([y[:, :HALF], onecol], axis=1)
        yhi[r] = jnp.concatenate([y[:, HALF:], onecol], axis=1)


def _k2_body(root1, alo, ahi, wrel, wroot, brgcn, root2, ylo, yhi):
    h = root1[...]
    for r in range(NREL):
        c = jnp.maximum(alo[r, :, HALF:HALF + 1], 1.0)
        s = jnp.concatenate([alo[r, :, :HALF], ahi[r, :, :HALF]], axis=1)
        h = h + s / c
    root2[...] = _dot(h, wroot[...]) + brgcn[...]
    for r in range(NREL):
        y = _dot(h, wrel[r])
        ylo[r] = y[:, :HALF]
        yhi[r] = y[:, HALF:]


def _k3_body(root2, blo, bhi, alo, wo1, bo1, wo2, bo2, out):
    h = root2[...]
    for r in range(NREL):
        c = jnp.maximum(alo[r, :, HALF:HALF + 1], 1.0)
        s = jnp.concatenate([blo[r], bhi[r]], axis=1)
        h = h + s / c
    z = _leaky(_dot(h, wo1[...]) + bo1[...])
    out[...] = _dot(z, wo2[...]) + bo2[...]


def _full(shape):
    nd = len(shape)
    return pl.BlockSpec(shape, lambda i: (0,) * nd)


def _rows(width):
    return pl.BlockSpec((BN, width), lambda i: (i, 0))


def _rel_rows(width):
    return pl.BlockSpec((NREL, BN, width), lambda i: (0, i, 0))


def _sds(shape):
    return jax.ShapeDtypeStruct(shape, jnp.float32)


def _k1(des, tweet, wd, bd, wt, bt, wi, bi, wrel, wroot, brgcn):
    return pl.pallas_call(
        _k1_body,
        grid=(NB,),
        in_specs=[
            _rows(768), _rows(768),
            _full((768, HALF)), _full((1, HALF)),
            _full((768, HALF)), _full((1, HALF)),
            _full((EMB, EMB)), _full((1, EMB)),
            _full((NREL, EMB, EMB)), _full((EMB, EMB)), _full((1, EMB)),
        ],
        out_specs=[_rel_rows(W1), _rel_rows(W1), _rows(EMB)],
        out_shape=[_sds((NREL, N, W1)), _sds((NREL, N, W1)),
                   _sds((N, EMB))],
    )(des, tweet, wd, bd, wt, bt, wi, bi, wrel, wroot, brgcn)


def _k2(root1, alo, ahi, wrel, wroot, brgcn):
    return pl.pallas_call(
        _k2_body,
        grid=(NB,),
        in_specs=[
            _rows(EMB), _rel_rows(W1), _rel_rows(W1),
            _full((NREL, EMB, EMB)), _full((EMB, EMB)), _full((1, EMB)),
        ],
        out_specs=[_rows(EMB), _rel_rows(W2), _rel_rows(W2)],
        out_shape=[_sds((N, EMB)), _sds((NREL, N, W2)),
                   _sds((NREL, N, W2))],
    )(root1, alo, ahi, wrel, wroot, brgcn)


def _k3(root2, blo, bhi, alo, wo1, bo1, wo2, bo2):
    return pl.pallas_call(
        _k3_body,
        grid=(NB,),
        in_specs=[
            _rows(EMB), _rel_rows(W2), _rel_rows(W2), _rel_rows(W1),
            _full((EMB, EMB)), _full((1, EMB)),
            _full((EMB, EMB)), _full((1, EMB)),
        ],
        out_specs=_rows(EMB),
        out_shape=_sds((N, EMB)),
    )(root2, blo, bhi, alo, wo1, bo1, wo2, bo2)


def _make_scatter(width):
    mesh = plsc.VectorSubcoreMesh(core_axis_name="c", subcore_axis_name="s")
    zpt = ACC_ROWS // NT

    @functools.partial(
        pl.kernel,
        mesh=mesh,
        compiler_params=pltpu.CompilerParams(use_tc_tiling_on_sc=False),
        out_type=[_sds((ACC_ROWS, width)), _sds((ACC_ROWS, width))],
        scratch_types=[
            pltpu.VMEM((CHUNK,), jnp.int32),
            pltpu.VMEM((CHUNK,), jnp.int32),
            pltpu.VMEM((CHUNK,), jnp.int32),
            pltpu.VMEM((CHUNK,), jnp.int32),
            pltpu.VMEM((CHUNK, width), jnp.float32),
            pltpu.VMEM((CHUNK, width), jnp.float32),
            pltpu.VMEM_SHARED((ACC_ROWS, width), jnp.float32),
            pltpu.SemaphoreType.DMA,
            pltpu.SemaphoreType.DMA,
        ],
    )
    def kern(gsrc, gdst, ylo, yhi, zrows, outlo, outhi,
             idx_a, dst_a, idx_b, dst_b, rows_a, rows_b, acc, sem_a, sem_b):
        cid = lax.axis_index("c")
        sid = lax.axis_index("s")

        pltpu.sync_copy(zrows, rows_a)
        for j in range(zpt // CHUNK):
            pltpu.sync_copy(rows_a, acc.at[pl.ds(sid * zpt + j * CHUNK,
                                                 CHUNK)])
        plsc.subcore_barrier()

        ebase = sid * (NCH * CHUNK)

        def edge_loop(tbl):
            pltpu.sync_copy(gsrc.at[pl.ds(ebase, CHUNK)], idx_a)
            pltpu.sync_copy(gdst.at[pl.ds(ebase, CHUNK)], dst_a)
            pltpu.async_copy(tbl.at[idx_a], rows_a, sem_a)
            pltpu.sync_copy(gsrc.at[pl.ds(ebase + CHUNK, CHUNK)], idx_b)
            pltpu.sync_copy(gdst.at[pl.ds(ebase + CHUNK, CHUNK)], dst_b)
            pltpu.async_copy(tbl.at[idx_b], rows_b, sem_b)

            def body(h, carry):
                off_a = ebase + (2 * h + 2) * CHUNK
                off_b = off_a + CHUNK
                pltpu.make_async_copy(tbl.at[idx_a], rows_a, sem_a).wait()
                pltpu.sync_copy(rows_a, acc.at[dst_a], add=True)
                pltpu.sync_copy(gsrc.at[pl.ds(off_a, CHUNK)], idx_a)
                pltpu.sync_copy(gdst.at[pl.ds(off_a, CHUNK)], dst_a)
                pltpu.make_async_copy(tbl.at[idx_b], rows_b, sem_b).wait()
                pltpu.sync_copy(rows_b, acc.at[dst_b], add=True)
                pltpu.sync_copy(gsrc.at[pl.ds(off_b, CHUNK)], idx_b)
                pltpu.sync_copy(gdst.at[pl.ds(off_b, CHUNK)], dst_b)
                pltpu.async_copy(tbl.at[idx_a], rows_a, sem_a)
                pltpu.async_copy(tbl.at[idx_b], rows_b, sem_b)
                return carry

            lax.fori_loop(0, NCH // 2, body, 0)
            pltpu.make_async_copy(tbl.at[idx_a], rows_a, sem_a).wait()
            pltpu.make_async_copy(tbl.at[idx_b], rows_b, sem_b).wait()

        @pl.when(cid == 0)
        def _():
            edge_loop(ylo)

        @pl.when(cid == 1)
        def _():
            edge_loop(yhi)

        plsc.subcore_barrier()
        dbase = sid * zpt

        def drain(out):
            for j in range(zpt // CHUNK):
                pltpu.sync_copy(acc.at[pl.ds(dbase + j * CHUNK, CHUNK)],
                                rows_a)
                pltpu.sync_copy(rows_a, out.at[pl.ds(dbase + j * CHUNK,
                                                     CHUNK)])

        @pl.when(cid == 0)
        def _():
            drain(outlo)

        @pl.when(cid == 1)
        def _():
            drain(outhi)

    return kern


_scatter1 = _make_scatter(W1)
_scatter2 = _make_scatter(W2)


def kernel(des, tweet, num_prop, cat_prop, edge_index, edge_type,
           Wd, bd, Wt, bt, Wi, bi, Wrel, Wroot, brgcn, Wo1, bo1, Wo2, bo2):
    del num_prop, cat_prop

    e_pad = NT * NCH * CHUNK + 2 * CHUNK
    et = edge_type.astype(jnp.int32)
    gsrc = et * N + edge_index[0]
    gdst = et * N + edge_index[1]
    pad = e_pad - gsrc.shape[0]
    gsrc = jnp.concatenate([gsrc, jnp.zeros((pad,), jnp.int32)])
    trash = TRASH + jnp.arange(pad, dtype=jnp.int32) % (ACC_ROWS - TRASH)
    gdst = jnp.concatenate([gdst, trash])
    zrows1 = jnp.zeros((CHUNK, W1), jnp.float32)
    zrows2 = jnp.zeros((CHUNK, W2), jnp.float32)

    bd2 = bd.reshape(1, HALF)
    bt2 = bt.reshape(1, HALF)
    bi2 = bi.reshape(1, EMB)
    brg2 = brgcn.reshape(1, EMB)
    bo12 = bo1.reshape(1, EMB)
    wo2p = jnp.pad(Wo2, ((0, 0), (0, EMB - Wo2.shape[1])))
    bo2p = jnp.pad(bo2, (0, EMB - bo2.shape[0])).reshape(1, EMB)

    ylo1, yhi1, root1 = _k1(des, tweet, Wd, bd2, Wt, bt2, Wi, bi2,
                            Wrel, Wroot, brg2)
    alo, ahi = _scatter1(gsrc, gdst,
                         ylo1.reshape(NREL * N, W1),
                         yhi1.reshape(NREL * N, W1), zrows1)
    alo = alo[:NREL * N].reshape(NREL, N, W1)
    ahi = ahi[:NREL * N].reshape(NREL, N, W1)

    root2, ylo2, yhi2 = _k2(root1, alo, ahi, Wrel, Wroot, brg2)
    blo, bhi = _scatter2(gsrc, gdst,
                         ylo2.reshape(NREL * N, W2),
                         yhi2.reshape(NREL * N, W2), zrows2)
    blo = blo[:NREL * N].reshape(NREL, N, W2)
    bhi = bhi[:NREL * N].reshape(NREL, N, W2)

    outp = _k3(root2, blo, bhi, alo, Wo1, bo12, wo2p, bo2p)
    return outp[:, :2]


# speedup vs baseline: 1.3152x; 1.3152x over previous
import functools

import jax
import jax.numpy as jnp
from jax import lax
from jax.experimental import pallas as pl
from jax.experimental.pallas import tpu as pltpu
from jax.experimental.pallas import tpu_sc as plsc

N = 10000
EMB = 128
HALF = 64
NREL = 2
NB = 5
BN = N // NB
CHUNK = 128
NT = 16
NCH = 158
ACC_ROWS = 20480
TRASH = 2 * N
W1 = 80
W2 = 64


def _leaky(x):
    return jnp.where(x >= 0, x, 0.01 * x)


def _dot(a, b):
    return jnp.dot(a, b, preferred_element_type=jnp.float32)


def _k1_body(des, tweet, wd, bd, wt, bt, wi, bi, wrel, wroot, brgcn,
             ylo, yhi, root):
    d = _leaky(_dot(des[...], wd[...]) + bd[...])
    t = _leaky(_dot(tweet[...], wt[...]) + bt[...])
    x = _leaky(_dot(jnp.concatenate([d, t], axis=1), wi[...]) + bi[...])
    root[...] = _dot(x, wroot[...]) + brgcn[...]
    onecol = (lax.broadcasted_iota(jnp.int32, (BN, W1 - HALF), 1) == 0
              ).astype(jnp.float32)
    for r in range(NREL):
        y = _dot(x, wrel[r])
        ylo[r] = jnp.concatenate([y[:, :HALF], onecol], axis=1)
        yhi[r] = jnp.concatenate([y[:, HALF:], onecol], axis=1)


def _k2_body(root1, alo, ahi, wrel, wroot, brgcn, root2, ylo, yhi):
    h = root1[...]
    for r in range(NREL):
        c = jnp.maximum(alo[r, :, HALF:HALF + 1], 1.0)
        s = jnp.concatenate([alo[r, :, :HALF], ahi[r, :, :HALF]], axis=1)
        h = h + s / c
    root2[...] = _dot(h, wroot[...]) + brgcn[...]
    for r in range(NREL):
        y = _dot(h, wrel[r])
        ylo[r] = y[:, :HALF]
        yhi[r] = y[:, HALF:]


def _k3_body(root2, blo, bhi, alo, wo1, bo1, wo2, bo2, out):
    h = root2[...]
    for r in range(NREL):
        c = jnp.maximum(alo[r, :, HALF:HALF + 1], 1.0)
        s = jnp.concatenate([blo[r], bhi[r]], axis=1)
        h = h + s / c
    z = _leaky(_dot(h, wo1[...]) + bo1[...])
    out[...] = _dot(z, wo2[...]) + bo2[...]


def _full(shape):
    nd = len(shape)
    return pl.BlockSpec(shape, lambda i: (0,) * nd)


def _rows(width):
    return pl.BlockSpec((BN, width), lambda i: (i, 0))


def _rel_rows(width):
    return pl.BlockSpec((NREL, BN, width), lambda i: (0, i, 0))


def _sds(shape):
    return jax.ShapeDtypeStruct(shape, jnp.float32)


def _k1(des, tweet, wd, bd, wt, bt, wi, bi, wrel, wroot, brgcn):
    return pl.pallas_call(
        _k1_body,
        grid=(NB,),
        in_specs=[
            _rows(768), _rows(768),
            _full((768, HALF)), _full((1, HALF)),
            _full((768, HALF)), _full((1, HALF)),
            _full((EMB, EMB)), _full((1, EMB)),
            _full((NREL, EMB, EMB)), _full((EMB, EMB)), _full((1, EMB)),
        ],
        out_specs=[_rel_rows(W1), _rel_rows(W1), _rows(EMB)],
        out_shape=[_sds((NREL, N, W1)), _sds((NREL, N, W1)),
                   _sds((N, EMB))],
    )(des, tweet, wd, bd, wt, bt, wi, bi, wrel, wroot, brgcn)


def _k2(root1, alo, ahi, wrel, wroot, brgcn):
    return pl.pallas_call(
        _k2_body,
        grid=(NB,),
        in_specs=[
            _rows(EMB), _rel_rows(W1), _rel_rows(W1),
            _full((NREL, EMB, EMB)), _full((EMB, EMB)), _full((1, EMB)),
        ],
        out_specs=[_rows(EMB), _rel_rows(W2), _rel_rows(W2)],
        out_shape=[_sds((N, EMB)), _sds((NREL, N, W2)),
                   _sds((NREL, N, W2))],
    )(root1, alo, ahi, wrel, wroot, brgcn)


def _k3(root2, blo, bhi, alo, wo1, bo1, wo2, bo2):
    return pl.pallas_call(
        _k3_body,
        grid=(NB,),
        in_specs=[
            _rows(EMB), _rel_rows(W2), _rel_rows(W2), _rel_rows(W1),
            _full((EMB, EMB)), _full((1, EMB)),
            _full((EMB, EMB)), _full((1, EMB)),
        ],
        out_specs=_rows(EMB),
        out_shape=_sds((N, EMB)),
    )(root2, blo, bhi, alo, wo1, bo1, wo2, bo2)


def _make_scatter(width):
    mesh = plsc.VectorSubcoreMesh(core_axis_name="c", subcore_axis_name="s")
    zpt = ACC_ROWS // NT

    @functools.partial(
        pl.kernel,
        mesh=mesh,
        compiler_params=pltpu.CompilerParams(use_tc_tiling_on_sc=False),
        out_type=[_sds((ACC_ROWS, width)), _sds((ACC_ROWS, width))],
        scratch_types=[
            pltpu.VMEM((2, CHUNK), jnp.int32),
            pltpu.VMEM((2, CHUNK), jnp.int32),
            pltpu.VMEM((CHUNK, width), jnp.float32),
            pltpu.VMEM((CHUNK, width), jnp.float32),
            pltpu.VMEM_SHARED((ACC_ROWS, width), jnp.float32),
            pltpu.SemaphoreType.DMA,
            pltpu.SemaphoreType.DMA,
        ],
    )
    def kern(gidx, ylo, yhi, zrows, outlo, outhi,
             iv_a, iv_b, rows_a, rows_b, acc, sem_a, sem_b):
        cid = lax.axis_index("c")
        sid = lax.axis_index("s")

        pltpu.sync_copy(zrows, rows_a)
        for j in range(zpt // CHUNK):
            pltpu.sync_copy(rows_a, acc.at[pl.ds(sid * zpt + j * CHUNK,
                                                 CHUNK)])
        plsc.subcore_barrier()

        bbase = sid * NCH

        def edge_loop(tbl):
            def body(h, carry):
                blk = bbase + 2 * h
                pltpu.sync_copy(gidx.at[blk], iv_a)
                ga = pltpu.async_copy(tbl.at[iv_a.at[0]], rows_a, sem_a)
                pltpu.sync_copy(gidx.at[blk + 1], iv_b)
                gb = pltpu.async_copy(tbl.at[iv_b.at[0]], rows_b, sem_b)
                ga.wait()
                pltpu.sync_copy(rows_a, acc.at[iv_a.at[1]], add=True)
                gb.wait()
                pltpu.sync_copy(rows_b, acc.at[iv_b.at[1]], add=True)
                return carry

            lax.fori_loop(0, NCH // 2, body, 0)

        @pl.when(cid == 0)
        def _():
            edge_loop(ylo)

        @pl.when(cid == 1)
        def _():
            edge_loop(yhi)

        plsc.subcore_barrier()
        dbase = sid * zpt

        def drain(out):
            for j in range(zpt // CHUNK):
                pltpu.sync_copy(acc.at[pl.ds(dbase + j * CHUNK, CHUNK)],
                                rows_a)
                pltpu.sync_copy(rows_a, out.at[pl.ds(dbase + j * CHUNK,
                                                     CHUNK)])

        @pl.when(cid == 0)
        def _():
            drain(outlo)

        @pl.when(cid == 1)
        def _():
            drain(outhi)

    return kern


_scatter1 = _make_scatter(W1)
_scatter2 = _make_scatter(W2)


def kernel(des, tweet, num_prop, cat_prop, edge_index, edge_type,
           Wd, bd, Wt, bt, Wi, bi, Wrel, Wroot, brgcn, Wo1, bo1, Wo2, bo2):
    del num_prop, cat_prop

    e_pad = NT * NCH * CHUNK
    et = edge_type.astype(jnp.int32)
    gsrc = et * N + edge_index[0]
    gdst = et * N + edge_index[1]
    pad = e_pad - gsrc.shape[0]
    gsrc = jnp.concatenate([gsrc, jnp.zeros((pad,), jnp.int32)])
    trash = TRASH + jnp.arange(pad, dtype=jnp.int32) % (ACC_ROWS - TRASH)
    gdst = jnp.concatenate([gdst, trash])
    gidx = jnp.stack([gsrc.reshape(NT * NCH, CHUNK),
                      gdst.reshape(NT * NCH, CHUNK)], axis=1)
    zrows1 = jnp.zeros((CHUNK, W1), jnp.float32)
    zrows2 = jnp.zeros((CHUNK, W2), jnp.float32)

    bd2 = bd.reshape(1, HALF)
    bt2 = bt.reshape(1, HALF)
    bi2 = bi.reshape(1, EMB)
    brg2 = brgcn.reshape(1, EMB)
    bo12 = bo1.reshape(1, EMB)
    wo2p = jnp.pad(Wo2, ((0, 0), (0, EMB - Wo2.shape[1])))
    bo2p = jnp.pad(bo2, (0, EMB - bo2.shape[0])).reshape(1, EMB)

    ylo1, yhi1, root1 = _k1(des, tweet, Wd, bd2, Wt, bt2, Wi, bi2,
                            Wrel, Wroot, brg2)
    alo, ahi = _scatter1(gidx,
                         ylo1.reshape(NREL * N, W1),
                         yhi1.reshape(NREL * N, W1), zrows1)
    alo = alo[:NREL * N].reshape(NREL, N, W1)
    ahi = ahi[:NREL * N].reshape(NREL, N, W1)

    root2, ylo2, yhi2 = _k2(root1, alo, ahi, Wrel, Wroot, brg2)
    blo, bhi = _scatter2(gidx,
                         ylo2.reshape(NREL * N, W2),
                         yhi2.reshape(NREL * N, W2), zrows2)
    blo = blo[:NREL * N].reshape(NREL, N, W2)
    bhi = bhi[:NREL * N].reshape(NREL, N, W2)

    outp = _k3(root2, blo, bhi, alo, Wo1, bo12, wo2p, bo2p)
    return outp[:, :2]
